# baseline (speedup 1.0000x reference)
import jax
import jax.numpy as jnp
from jax.experimental import pallas as pl
from jax.experimental.pallas import tpu as pltpu

def _body(x_ref, o_ref):
    o_ref[...] = x_ref[...]

def kernel(context, forecast, forecast_mask, step, W1, b1, W2, b2, pos_emb):
    w = W2 + 1.0
    out = pl.pallas_call(
        _body,
        in_specs=[pl.BlockSpec((8, 128), lambda: (0, 0))],
        out_specs=pl.BlockSpec((8, 128), lambda: (0, 0)),
        out_shape=jax.ShapeDtypeStruct((8, 128), jnp.float32),
    )(w[:8, :128])
    return (out, w, out)


# speedup vs baseline: 1.4022x; 1.4022x over previous
import functools

import jax
import jax.numpy as jnp
from jax.experimental import pallas as pl
from jax.experimental.pallas import tpu as pltpu

_KT = 4096


def _body(K, KT, c_ref, f_ref, m_ref, w1_ref, b1_ref, pe_ref, w2_ref, b2_ref,
          logits_ref, probs_ref, act_ref, lbuf_ref, h_ref, rmax_ref, rsum_ref,
          rarg_ref, lse_ref):
    j = pl.program_id(0)
    nk = pl.num_programs(0) // 2

    @pl.when(j == 0)
    def _init():
        m = m_ref[...]
        x = jnp.concatenate([c_ref[...], f_ref[...] * m, m], axis=-1)
        h = jnp.dot(x, w1_ref[...], preferred_element_type=jnp.float32)
        h = h + b1_ref[...] + pe_ref[...]
        h_ref[...] = jnp.maximum(h, 0.0)
        rmax_ref[...] = jnp.full_like(rmax_ref, -jnp.inf)
        rsum_ref[...] = jnp.zeros_like(rsum_ref)
        rarg_ref[...] = jnp.zeros_like(rarg_ref)

    @pl.when(j < nk)
    def _phase_a():
        k = j
        logits = jnp.dot(h_ref[...], w2_ref[...],
                         preferred_element_type=jnp.float32) + b2_ref[...]
        logits_ref[...] = logits
        lbuf_ref[:, pl.ds(k * KT, KT)] = logits

        col0 = k * KT
        col = jax.lax.broadcasted_iota(jnp.int32, logits.shape, 1)
        valid = (col0 + col) < K
        masked = jnp.where((k < nk - 1) | valid, logits, -jnp.inf)

        tmax = jnp.max(masked, axis=-1, keepdims=True)
        targ = jnp.argmax(masked, axis=-1).astype(jnp.int32)[:, None] + col0

        rmax = rmax_ref[...]
        new_max = jnp.maximum(rmax, tmax)
        tsum = jnp.sum(jnp.exp(masked - new_max), axis=-1, keepdims=True)
        rsum_ref[...] = rsum_ref[...] * jnp.exp(rmax - new_max) + tsum
        rarg = rarg_ref[...]
        rarg_ref[...] = jnp.where(
            tmax > rmax, targ,
            jnp.where(tmax == rmax, jnp.minimum(targ, rarg), rarg))
        rmax_ref[...] = new_max

        @pl.when(k == nk - 1)
        def _fin():
            act_ref[...] = rarg_ref[...].astype(jnp.float32)
            lse_ref[...] = rmax_ref[...] + jnp.log(rsum_ref[...])

    @pl.when(j >= nk)
    def _phase_b():
        k = j - nk
        probs_ref[...] = jnp.exp(lbuf_ref[:, pl.ds(k * KT, KT)] - lse_ref[...])


def kernel(context, forecast, forecast_mask, step, W1, b1, W2, b2, pos_emb):
    B, L = context.shape
    H = forecast.shape[1]
    D = W1.shape[1]
    K = W2.shape[1]
    KT = _KT
    nk = pl.cdiv(K, KT)
    Kpad = nk * KT

    m = forecast_mask.astype(jnp.float32)
    pe = jax.lax.dynamic_index_in_dim(pos_emb, step, axis=0, keepdims=True)
    b1_2d = b1.reshape(1, D)
    b2_2d = b2.reshape(1, K)

    last = nk - 1

    logits, probs, act = pl.pallas_call(
        functools.partial(_body, K, KT),
        grid=(2 * nk,),
        in_specs=[
            pl.BlockSpec((B, L), lambda j: (0, 0)),
            pl.BlockSpec((B, H), lambda j: (0, 0)),
            pl.BlockSpec((B, H), lambda j: (0, 0)),
            pl.BlockSpec((L + 2 * H, D), lambda j: (0, 0)),
            pl.BlockSpec((1, D), lambda j: (0, 0)),
            pl.BlockSpec((1, D), lambda j: (0, 0)),
            pl.BlockSpec((D, KT), lambda j: (0, jnp.minimum(j, last))),
            pl.BlockSpec((1, KT), lambda j: (0, jnp.minimum(j, last))),
        ],
        out_specs=[
            pl.BlockSpec((B, KT), lambda j: (0, jnp.minimum(j, last))),
            pl.BlockSpec((B, KT),
                         lambda j: (0, jnp.maximum(j - (last + 1), 0))),
            pl.BlockSpec((B, 1), lambda j: (0, 0)),
        ],
        out_shape=[
            jax.ShapeDtypeStruct((B, K), jnp.float32),
            jax.ShapeDtypeStruct((B, K), jnp.float32),
            jax.ShapeDtypeStruct((B, 1), jnp.float32),
        ],
        scratch_shapes=[
            pltpu.VMEM((B, Kpad), jnp.float32),
            pltpu.VMEM((B, D), jnp.float32),
            pltpu.VMEM((B, 1), jnp.float32),
            pltpu.VMEM((B, 1), jnp.float32),
            pltpu.VMEM((B, 1), jnp.int32),
            pltpu.VMEM((B, 1), jnp.float32),
        ],
        compiler_params=pltpu.CompilerParams(
            dimension_semantics=("arbitrary",)),
    )(context, forecast, m, W1, b1_2d, pe, W2, b2_2d)

    return (act[:, 0], probs, logits)
